# baseline (speedup 1.0000x reference)
import functools

import jax
import jax.numpy as jnp
from jax import lax
from jax.experimental import pallas as pl
from jax.experimental.pallas import tpu as pltpu
from jax.experimental.pallas import tpu_sc as plsc

_NC = 2
_NS = 16
_CH = 128
_L = 16
_ZR = 128
_HQ = 32
_NB = 10


def _sc_mesh():
    return plsc.VectorSubcoreMesh(core_axis_name="c", subcore_axis_name="s")


def _make_deg_kernel(nch, nrows):
    ncd = nch // 2
    rpt = nrows // _NS
    nz = rpt // _ZR

    @functools.partial(
        pl.kernel,
        out_type=jax.ShapeDtypeStruct((_NC, nrows, _L), jnp.float32),
        mesh=_sc_mesh(),
        scratch_types=[
            pltpu.VMEM((ncd, _CH), jnp.int32),
            pltpu.VMEM((_CH, _L), jnp.float32),
            pltpu.VMEM((_ZR, _L), jnp.float32),
            pltpu.VMEM_SHARED((nrows, _L), jnp.float32),
        ],
        compiler_params=pltpu.CompilerParams(use_tc_tiling_on_sc=False),
    )
    def deg_kernel(dst_hbm, out_hbm, didx, ones_v, zb, acc):
        cid = lax.axis_index("c")
        sid = lax.axis_index("s")
        pltpu.sync_copy(dst_hbm.at[sid, pl.ds(cid * ncd, ncd)], didx)

        @pl.loop(0, _CH)
        def _(r):
            ones_v[r, :] = jnp.ones((_L,), jnp.float32)

        @pl.loop(0, _ZR)
        def _(r):
            zb[r, :] = jnp.zeros((_L,), jnp.float32)

        base = sid * rpt
        for z in range(nz):
            pltpu.sync_copy(zb, acc.at[pl.ds(base + z * _ZR, _ZR), :])
        plsc.subcore_barrier()

        @pl.loop(0, ncd)
        def _(c):
            pltpu.sync_copy(ones_v, acc.at[didx.at[c]], add=True)

        plsc.subcore_barrier()
        for z in range(nz):
            pltpu.sync_copy(acc.at[pl.ds(base + z * _ZR, _ZR), :],
                            out_hbm.at[cid, pl.ds(base + z * _ZR, _ZR), :])

    return deg_kernel


def _make_scatter_kernel(nch, nrows, n, h):
    npass = h // _HQ
    rpt = nrows // _NS
    nz = rpt // _ZR
    lrows = n // _NS

    @functools.partial(
        pl.kernel,
        out_type=jax.ShapeDtypeStruct((_NC, nrows, h), jnp.float32),
        mesh=_sc_mesh(),
        scratch_types=[
            pltpu.VMEM((nch, _CH), jnp.int32),
            pltpu.VMEM((nch, _CH), jnp.int32),
            [pltpu.VMEM((_CH, _HQ), jnp.float32) for _ in range(_NB)],
            pltpu.VMEM((_ZR, _HQ), jnp.float32),
            pltpu.VMEM_SHARED((nrows, _HQ), jnp.float32),
            pltpu.VMEM_SHARED((nrows, _HQ), jnp.float32),
            [pltpu.SemaphoreType.DMA for _ in range(_NB)],
            [pltpu.SemaphoreType.DMA for _ in range(_NB)],
        ],
        compiler_params=pltpu.CompilerParams(use_tc_tiling_on_sc=False),
    )
    def scat_kernel(ya_hbm, yb_hbm, src_hbm, dst_hbm, out_hbm,
                    sidx, didx, gbufs, zb, ytab, acc, gsems, ssems):
        cid = lax.axis_index("c")
        sid = lax.axis_index("s")
        pltpu.sync_copy(src_hbm.at[sid], sidx)
        pltpu.sync_copy(dst_hbm.at[sid], didx)

        @pl.loop(0, _ZR)
        def _(r):
            for cc in range(_HQ // _L):
                zb[r, pl.ds(cc * _L, _L)] = jnp.zeros((_L,), jnp.float32)

        base = sid * rpt
        lbase = sid * lrows
        for p in range(npass):
            for z in range(nz):
                pltpu.sync_copy(zb, acc.at[pl.ds(base + z * _ZR, _ZR), :])

            @pl.when(cid == 0)
            def _(p=p):
                pltpu.sync_copy(
                    ya_hbm.at[pl.ds(lbase, lrows), pl.ds(p * _HQ, _HQ)],
                    ytab.at[pl.ds(lbase, lrows), :])

            @pl.when(cid == 1)
            def _(p=p):
                pltpu.sync_copy(
                    yb_hbm.at[pl.ds(lbase, lrows), pl.ds(p * _HQ, _HQ)],
                    ytab.at[pl.ds(lbase, lrows), :])

            plsc.subcore_barrier()

            for b in range(_NB):
                pltpu.async_copy(ytab.at[sidx.at[b]], gbufs[b], gsems[b])

            @pl.loop(0, nch, step=_NB)
            def _(c):
                for b in range(_NB):
                    pltpu.make_async_copy(
                        ytab.at[sidx.at[c + b]], gbufs[b], gsems[b]).wait()
                    pltpu.async_copy(
                        gbufs[b], acc.at[didx.at[c + b]], ssems[b], add=True)
                for b in range(_NB):
                    @pl.when(c + b + _NB < nch)
                    def _(b=b):
                        pltpu.make_async_copy(
                            gbufs[b], acc.at[didx.at[c + b]], ssems[b]).wait()
                        pltpu.async_copy(
                            ytab.at[sidx.at[c + b + _NB]], gbufs[b], gsems[b])

            for b in range(_NB):
                pltpu.make_async_copy(
                    gbufs[b], acc.at[didx.at[nch - _NB + b]], ssems[b]).wait()

            plsc.subcore_barrier()
            for z in range(nz):
                pltpu.sync_copy(
                    acc.at[pl.ds(base + z * _ZR, _ZR), :],
                    out_hbm.at[cid, pl.ds(base + z * _ZR, _ZR),
                               pl.ds(p * _HQ, _HQ)])

    return scat_kernel


def _stage_mm(x, w1):
    n = x.shape[0]

    def body(x_ref, w_ref, xw_ref):
        xw_ref[...] = jnp.dot(x_ref[...], w_ref[...],
                              preferred_element_type=jnp.float32)

    return pl.pallas_call(
        body,
        out_shape=jax.ShapeDtypeStruct((n, w1.shape[1]), jnp.float32),
    )(x, w1)


def _stage_norm(deg2, xw):
    n, h = xw.shape
    hh = h // 2

    def body(deg_ref, xw_ref, ya_ref, yb_ref, dinv_ref):
        dv = deg_ref[...]
        d = dv[0, :n, 0:1] + dv[1, :n, 0:1] + 1.0
        dinv = lax.rsqrt(d)
        y = xw_ref[...] * dinv
        ya_ref[...] = y[:, :hh]
        yb_ref[...] = y[:, hh:]
        dinv_ref[...] = dinv

    return pl.pallas_call(
        body,
        out_shape=(
            jax.ShapeDtypeStruct((n, hh), jnp.float32),
            jax.ShapeDtypeStruct((n, hh), jnp.float32),
            jax.ShapeDtypeStruct((n, 1), jnp.float32),
        ),
    )(deg2, xw)


def _stage_mid(tmp, ya, yb, dinv, b, g, be, w):
    n, hh = ya.shape
    hout = w.shape[1]

    def body(t_ref, ya_ref, yb_ref, dinv_ref, b_ref, g_ref, be_ref, w_ref,
             oa_ref, ob_ref):
        t = t_ref[...]
        dinv = dinv_ref[...]

        def half(agg, sl):
            hpre = agg * dinv + b_ref[0:1, sl]
            m = jnp.mean(hpre, axis=0, keepdims=True)
            cen = hpre - m
            v = jnp.mean(cen * cen, axis=0, keepdims=True)
            hn = cen * lax.rsqrt(v + 1e-5) * g_ref[0:1, sl] + be_ref[0:1, sl]
            return jnp.maximum(hn, 0.0)

        ha = half(t[0, :n] + ya_ref[...], slice(0, hh))
        hb = half(t[1, :n] + yb_ref[...], slice(hh, 2 * hh))
        wv = w_ref[...]
        o = (jnp.dot(ha, wv[:hh], preferred_element_type=jnp.float32)
             + jnp.dot(hb, wv[hh:], preferred_element_type=jnp.float32)) * dinv
        oa_ref[...] = o[:, :hout // 2]
        ob_ref[...] = o[:, hout // 2:]

    return pl.pallas_call(
        body,
        out_shape=(
            jax.ShapeDtypeStruct((n, hout // 2), jnp.float32),
            jax.ShapeDtypeStruct((n, hout // 2), jnp.float32),
        ),
    )(tmp, ya, yb, dinv, b, g, be, w)


def _stage_final(tmp, ya, yb, dinv, b):
    n, hh = ya.shape

    def body(t_ref, ya_ref, yb_ref, dinv_ref, b_ref, oa_ref, ob_ref):
        t = t_ref[...]
        dinv = dinv_ref[...]
        oa_ref[...] = (t[0, :n] + ya_ref[...]) * dinv + b_ref[0:1, :hh]
        ob_ref[...] = (t[1, :n] + yb_ref[...]) * dinv + b_ref[0:1, hh:]

    return pl.pallas_call(
        body,
        out_shape=(
            jax.ShapeDtypeStruct((n, hh), jnp.float32),
            jax.ShapeDtypeStruct((n, hh), jnp.float32),
        ),
    )(tmp, ya, yb, dinv, b)


def kernel(x, edge_index, W1, b1, g1, be1, W2, b2, g2, be2, W3, b3):
    n, _ = x.shape
    e = edge_index.shape[1]
    h = W1.shape[1]
    c = W3.shape[1]

    nch = -(-e // (_NS * _CH))
    nch = -(-nch // (8 * _NB)) * (8 * _NB)
    e_pad = _NS * nch * _CH
    pad = e_pad - e
    src = jnp.concatenate(
        [edge_index[0], jnp.zeros((pad,), jnp.int32)]).reshape(_NS, nch, _CH)
    dst = jnp.concatenate(
        [edge_index[1], jnp.full((pad,), n, jnp.int32)]).reshape(_NS, nch, _CH)

    nrows = -(-(n + 1) // (_NS * _ZR)) * (_NS * _ZR)

    cp = -(-c // (2 * _L)) * (2 * _L)
    w3p = jnp.pad(W3, ((0, 0), (0, cp - c)))
    b3p = jnp.pad(b3, (0, cp - c)).reshape(1, cp)

    npad = -(-n // _NS) * _NS
    rp = npad - n

    def padr(t):
        return jnp.pad(t, ((0, rp), (0, 0))) if rp else t

    deg2 = _make_deg_kernel(nch, nrows)(dst)
    xw1 = _stage_mm(x, W1)
    ya1, yb1, dinv = _stage_norm(deg2, xw1)
    tmp1 = _make_scatter_kernel(nch, nrows, npad, h // 2)(
        padr(ya1), padr(yb1), src, dst)
    ya2, yb2 = _stage_mid(tmp1, ya1, yb1, dinv, b1.reshape(1, -1),
                          g1.reshape(1, -1), be1.reshape(1, -1), W2)
    tmp2 = _make_scatter_kernel(nch, nrows, npad, h // 2)(
        padr(ya2), padr(yb2), src, dst)
    ya3, yb3 = _stage_mid(tmp2, ya2, yb2, dinv, b2.reshape(1, -1),
                          g2.reshape(1, -1), be2.reshape(1, -1), w3p)
    tmp3 = _make_scatter_kernel(nch, nrows, npad, cp // 2)(
        padr(ya3), padr(yb3), src, dst)
    oa, ob = _stage_final(tmp3, ya3, yb3, dinv, b3p)
    return jnp.concatenate([oa, ob], axis=1)[:, :c]


# speedup vs baseline: 1.0141x; 1.0141x over previous
import functools

import jax
import jax.numpy as jnp
from jax import lax
from jax.experimental import pallas as pl
from jax.experimental.pallas import tpu as pltpu
from jax.experimental.pallas import tpu_sc as plsc

_NC = 2
_NS = 16
_CH = 128
_L = 16
_ZR = 128
_HQ = 32
_NB = 10


def _sc_mesh():
    return plsc.VectorSubcoreMesh(core_axis_name="c", subcore_axis_name="s")


def _make_deg_kernel(nch, nrows):
    ncd = nch // 2
    rpt = nrows // _NS
    nz = rpt // _ZR

    @functools.partial(
        pl.kernel,
        out_type=jax.ShapeDtypeStruct((_NC, nrows, _L), jnp.float32),
        mesh=_sc_mesh(),
        scratch_types=[
            pltpu.VMEM((ncd, _CH), jnp.int32),
            pltpu.VMEM((_CH, _L), jnp.float32),
            pltpu.VMEM((_ZR, _L), jnp.float32),
            pltpu.VMEM((_ZR, _L), jnp.float32),
            pltpu.VMEM_SHARED((nrows, _L), jnp.float32),
        ],
        compiler_params=pltpu.CompilerParams(use_tc_tiling_on_sc=False),
    )
    def deg_kernel(dst_hbm, out_hbm, didx, ones_v, zb, ob, acc):
        cid = lax.axis_index("c")
        sid = lax.axis_index("s")
        pltpu.sync_copy(dst_hbm.at[sid, pl.ds(cid * ncd, ncd)], didx)

        @pl.loop(0, _CH)
        def _(r):
            ones_v[r, :] = jnp.ones((_L,), jnp.float32)

        @pl.loop(0, _ZR)
        def _(r):
            zb[r, :] = jnp.zeros((_L,), jnp.float32)

        base = sid * rpt
        for z in range(nz):
            pltpu.sync_copy(zb, acc.at[pl.ds(base + z * _ZR, _ZR), :])
        plsc.subcore_barrier()

        @pl.loop(0, ncd)
        def _(c):
            pltpu.sync_copy(ones_v, acc.at[didx.at[c]], add=True)

        plsc.subcore_barrier()
        for z in range(nz):
            pltpu.sync_copy(acc.at[pl.ds(base + z * _ZR, _ZR), :], ob)
            pltpu.sync_copy(ob, out_hbm.at[cid, pl.ds(base + z * _ZR, _ZR), :])

    return deg_kernel


def _make_scatter_kernel(nch, nrows, n, h):
    npass = h // _HQ
    rpt = nrows // _NS
    nz = rpt // _ZR
    lrows = n // _NS

    @functools.partial(
        pl.kernel,
        out_type=jax.ShapeDtypeStruct((_NC, nrows, h), jnp.float32),
        mesh=_sc_mesh(),
        scratch_types=[
            pltpu.VMEM((nch, _CH), jnp.int32),
            pltpu.VMEM((nch, _CH), jnp.int32),
            [pltpu.VMEM((_CH, _HQ), jnp.float32) for _ in range(_NB)],
            pltpu.VMEM((_ZR, _HQ), jnp.float32),
            pltpu.VMEM((_ZR, _HQ), jnp.float32),
            pltpu.VMEM_SHARED((nrows, _HQ), jnp.float32),
            pltpu.VMEM_SHARED((nrows, _HQ), jnp.float32),
            [pltpu.SemaphoreType.DMA for _ in range(_NB)],
            [pltpu.SemaphoreType.DMA for _ in range(_NB)],
        ],
        compiler_params=pltpu.CompilerParams(use_tc_tiling_on_sc=False),
    )
    def scat_kernel(ya_hbm, yb_hbm, src_hbm, dst_hbm, out_hbm,
                    sidx, didx, gbufs, zb, ob, ytab, acc, gsems, ssems):
        cid = lax.axis_index("c")
        sid = lax.axis_index("s")
        pltpu.sync_copy(src_hbm.at[sid], sidx)
        pltpu.sync_copy(dst_hbm.at[sid], didx)

        @pl.loop(0, _ZR)
        def _(r):
            for cc in range(_HQ // _L):
                zb[r, pl.ds(cc * _L, _L)] = jnp.zeros((_L,), jnp.float32)

        base = sid * rpt
        lbase = sid * lrows
        for p in range(npass):
            for z in range(nz):
                pltpu.sync_copy(zb, acc.at[pl.ds(base + z * _ZR, _ZR), :])

            @pl.when(cid == 0)
            def _(p=p):
                pltpu.sync_copy(
                    ya_hbm.at[pl.ds(lbase, lrows), pl.ds(p * _HQ, _HQ)],
                    ytab.at[pl.ds(lbase, lrows), :])

            @pl.when(cid == 1)
            def _(p=p):
                pltpu.sync_copy(
                    yb_hbm.at[pl.ds(lbase, lrows), pl.ds(p * _HQ, _HQ)],
                    ytab.at[pl.ds(lbase, lrows), :])

            plsc.subcore_barrier()

            for b in range(_NB):
                pltpu.async_copy(ytab.at[sidx.at[b]], gbufs[b], gsems[b])

            @pl.loop(0, nch, step=_NB)
            def _(c):
                for b in range(_NB):
                    pltpu.make_async_copy(
                        ytab.at[sidx.at[c + b]], gbufs[b], gsems[b]).wait()
                    pltpu.async_copy(
                        gbufs[b], acc.at[didx.at[c + b]], ssems[b], add=True)
                for b in range(_NB):
                    @pl.when(c + b + _NB < nch)
                    def _(b=b):
                        pltpu.make_async_copy(
                            gbufs[b], acc.at[didx.at[c + b]], ssems[b]).wait()
                        pltpu.async_copy(
                            ytab.at[sidx.at[c + b + _NB]], gbufs[b], gsems[b])

            for b in range(_NB):
                pltpu.make_async_copy(
                    gbufs[b], acc.at[didx.at[nch - _NB + b]], ssems[b]).wait()

            plsc.subcore_barrier()
            for z in range(nz):
                pltpu.sync_copy(acc.at[pl.ds(base + z * _ZR, _ZR), :], ob)
                pltpu.sync_copy(
                    ob, out_hbm.at[cid, pl.ds(base + z * _ZR, _ZR),
                                   pl.ds(p * _HQ, _HQ)])
            if p + 1 < npass:
                plsc.subcore_barrier()

    return scat_kernel


def _stage_mm(x, w1):
    n = x.shape[0]

    def body(x_ref, w_ref, xw_ref):
        xw_ref[...] = jnp.dot(x_ref[...], w_ref[...],
                              preferred_element_type=jnp.float32)

    return pl.pallas_call(
        body,
        out_shape=jax.ShapeDtypeStruct((n, w1.shape[1]), jnp.float32),
    )(x, w1)


def _stage_norm(deg2, xw):
    n, h = xw.shape
    hh = h // 2

    def body(deg_ref, xw_ref, ya_ref, yb_ref, dinv_ref):
        dv = deg_ref[...]
        d = dv[0, :n, 0:1] + dv[1, :n, 0:1] + 1.0
        dinv = lax.rsqrt(d)
        y = xw_ref[...] * dinv
        ya_ref[...] = y[:, :hh]
        yb_ref[...] = y[:, hh:]
        dinv_ref[...] = dinv

    return pl.pallas_call(
        body,
        out_shape=(
            jax.ShapeDtypeStruct((n, hh), jnp.float32),
            jax.ShapeDtypeStruct((n, hh), jnp.float32),
            jax.ShapeDtypeStruct((n, 1), jnp.float32),
        ),
    )(deg2, xw)


def _stage_mid(tmp, ya, yb, dinv, b, g, be, w):
    n, hh = ya.shape
    hout = w.shape[1]

    def body(t_ref, ya_ref, yb_ref, dinv_ref, b_ref, g_ref, be_ref, w_ref,
             oa_ref, ob_ref):
        t = t_ref[...]
        dinv = dinv_ref[...]

        def half(agg, sl):
            hpre = agg * dinv + b_ref[0:1, sl]
            m = jnp.mean(hpre, axis=0, keepdims=True)
            cen = hpre - m
            v = jnp.mean(cen * cen, axis=0, keepdims=True)
            hn = cen * lax.rsqrt(v + 1e-5) * g_ref[0:1, sl] + be_ref[0:1, sl]
            return jnp.maximum(hn, 0.0)

        ha = half(t[0, :n] + ya_ref[...], slice(0, hh))
        hb = half(t[1, :n] + yb_ref[...], slice(hh, 2 * hh))
        wv = w_ref[...]
        o = (jnp.dot(ha, wv[:hh], preferred_element_type=jnp.float32)
             + jnp.dot(hb, wv[hh:], preferred_element_type=jnp.float32)) * dinv
        oa_ref[...] = o[:, :hout // 2]
        ob_ref[...] = o[:, hout // 2:]

    return pl.pallas_call(
        body,
        out_shape=(
            jax.ShapeDtypeStruct((n, hout // 2), jnp.float32),
            jax.ShapeDtypeStruct((n, hout // 2), jnp.float32),
        ),
    )(tmp, ya, yb, dinv, b, g, be, w)


def _stage_final(tmp, ya, yb, dinv, b):
    n, hh = ya.shape

    def body(t_ref, ya_ref, yb_ref, dinv_ref, b_ref, oa_ref, ob_ref):
        t = t_ref[...]
        dinv = dinv_ref[...]
        oa_ref[...] = (t[0, :n] + ya_ref[...]) * dinv + b_ref[0:1, :hh]
        ob_ref[...] = (t[1, :n] + yb_ref[...]) * dinv + b_ref[0:1, hh:]

    return pl.pallas_call(
        body,
        out_shape=(
            jax.ShapeDtypeStruct((n, hh), jnp.float32),
            jax.ShapeDtypeStruct((n, hh), jnp.float32),
        ),
    )(tmp, ya, yb, dinv, b)


def kernel(x, edge_index, W1, b1, g1, be1, W2, b2, g2, be2, W3, b3):
    n, _ = x.shape
    e = edge_index.shape[1]
    h = W1.shape[1]
    c = W3.shape[1]

    nch = -(-e // (_NS * _CH))
    nch = -(-nch // (8 * _NB)) * (8 * _NB)
    e_pad = _NS * nch * _CH
    pad = e_pad - e
    src = jnp.concatenate(
        [edge_index[0], jnp.zeros((pad,), jnp.int32)]).reshape(_NS, nch, _CH)
    dst = jnp.concatenate(
        [edge_index[1], jnp.full((pad,), n, jnp.int32)]).reshape(_NS, nch, _CH)

    nrows = -(-(n + 1) // (_NS * _ZR)) * (_NS * _ZR)

    cp = -(-c // (2 * _L)) * (2 * _L)
    w3p = jnp.pad(W3, ((0, 0), (0, cp - c)))
    b3p = jnp.pad(b3, (0, cp - c)).reshape(1, cp)

    npad = -(-n // _NS) * _NS
    rp = npad - n

    def padr(t):
        return jnp.pad(t, ((0, rp), (0, 0))) if rp else t

    deg2 = _make_deg_kernel(nch, nrows)(dst)
    xw1 = _stage_mm(x, W1)
    ya1, yb1, dinv = _stage_norm(deg2, xw1)
    tmp1 = _make_scatter_kernel(nch, nrows, npad, h // 2)(
        padr(ya1), padr(yb1), src, dst)
    ya2, yb2 = _stage_mid(tmp1, ya1, yb1, dinv, b1.reshape(1, -1),
                          g1.reshape(1, -1), be1.reshape(1, -1), W2)
    tmp2 = _make_scatter_kernel(nch, nrows, npad, h // 2)(
        padr(ya2), padr(yb2), src, dst)
    ya3, yb3 = _stage_mid(tmp2, ya2, yb2, dinv, b2.reshape(1, -1),
                          g2.reshape(1, -1), be2.reshape(1, -1), w3p)
    tmp3 = _make_scatter_kernel(nch, nrows, npad, cp // 2)(
        padr(ya3), padr(yb3), src, dst)
    oa, ob = _stage_final(tmp3, ya3, yb3, dinv, b3p)
    return jnp.concatenate([oa, ob], axis=1)[:, :c]
